# baseline (speedup 1.0000x reference)
import functools

import jax
import jax.numpy as jnp
from jax import lax
from jax.experimental import pallas as pl
from jax.experimental.pallas import tpu as pltpu
from jax.experimental.pallas import tpu_sc as plsc

N = 10000
K = 16
F = 256

BQ = 16
NPAD = 10112
NLPT = 4
_IMAX = 0x7FFFFFFF


def _knn_body(posq_ref, post_ref, out_ref):
    q = posq_ref[...]
    pt = post_ref[...]
    sq = jnp.sum(pt * pt, axis=0, keepdims=True)
    qq = jnp.sum(q * q, axis=1, keepdims=True)
    d2 = qq - 2.0 * lax.dot_general(q, pt, (((1,), (0,)), ((), ())),
                                    preferred_element_type=jnp.float32) + sq
    iota = lax.broadcasted_iota(jnp.int32, (BQ, NPAD), 1)
    bits = lax.bitcast_convert_type(jnp.maximum(d2, 0.0), jnp.int32)
    key = (bits & jnp.int32(-16384)) | iota

    tops = [[jnp.full((BQ, 128), jnp.int32(_IMAX), jnp.int32)
             for _ in range(NLPT)] for _ in range(2)]
    for c in range(NPAD // 128):
        x = key[:, c * 128:(c + 1) * 128]
        tset = tops[c % 2]
        for i in range(NLPT):
            ti = tset[i]
            lo = jnp.minimum(ti, x)
            x = jnp.maximum(ti, x)
            tset[i] = lo

    cand = jnp.concatenate(tops[0] + tops[1], axis=1)
    cols = []
    for _ in range(K):
        m = jnp.min(cand, axis=1, keepdims=True)
        cols.append(m & jnp.int32(16383))
        cand = jnp.where(cand == m, jnp.int32(_IMAX), cand)
    out_ref[...] = jnp.concatenate(cols, axis=1)

    v = m
    tlast = jnp.minimum(tops[0][NLPT - 1], tops[1][NLPT - 1])
    flag = jnp.any(jnp.min(tlast, axis=1, keepdims=True) < v)

    @pl.when(flag)
    def _slow():
        kk = key
        cols2 = []
        for _ in range(K):
            mm = jnp.min(kk, axis=1, keepdims=True)
            cols2.append(mm & jnp.int32(16383))
            kk = jnp.where(kk == mm, jnp.int32(_IMAX), kk)
        out_ref[...] = jnp.concatenate(cols2, axis=1)


def _knn(pos, post):
    return pl.pallas_call(
        _knn_body,
        grid=(N // BQ,),
        in_specs=[
            pl.BlockSpec((BQ, 3), lambda i: (i, 0)),
            pl.BlockSpec((3, NPAD), lambda i: (0, 0)),
        ],
        out_specs=pl.BlockSpec((BQ, K), lambda i: (i, 0)),
        out_shape=jax.ShapeDtypeStruct((N, K), jnp.int32),
    )(pos, post)


def _pack_bf16(g):
    ge = g[:, :128].astype(jnp.bfloat16).astype(jnp.float32)
    go = g[:, 128:].astype(jnp.bfloat16).astype(jnp.float32)
    eb = lax.bitcast_convert_type(ge, jnp.int32)
    ob = lax.bitcast_convert_type(go, jnp.int32)
    return lax.shift_right_logical(eb, 16) | (ob & jnp.int32(-65536))


def _unpack_bf16(w):
    even = lax.bitcast_convert_type(lax.shift_left(w, 16), jnp.float32)
    odd = lax.bitcast_convert_type(w & jnp.int32(-65536), jnp.float32)
    return jnp.concatenate([even, odd], axis=1)


BT = 400


def _tables_body(pos_ref, w1_ref, w2p_ref, w3p_ref, ba1_ref, ba2_ref, ba3_ref,
                 g1_ref, c1_ref, b2_ref, c2_ref, b3_ref, c3_ref):
    p = pos_ref[...]

    def mm(w):
        return lax.dot_general(p, w, (((1,), (0,)), ((), ())),
                               preferred_element_type=jnp.float32)

    w1 = w1_ref[...]
    b1 = mm(w1[3:6, :])
    g1_ref[...] = _pack_bf16(mm(w1[0:3, :]) + b1)
    c1_ref[...] = b1 - ba1_ref[...]
    b2 = mm(w2p_ref[...])
    b2_ref[...] = b2
    c2_ref[...] = b2 - ba2_ref[...]
    b3 = mm(w3p_ref[...])
    b3_ref[...] = b3
    c3_ref[...] = b3 - ba3_ref[...]


def _tables(pos, w1a, w2p, w3p, ba1, ba2, ba3):
    outf = jax.ShapeDtypeStruct((N, F), jnp.float32)
    outb = jax.ShapeDtypeStruct((N, F // 2), jnp.int32)
    return pl.pallas_call(
        _tables_body,
        grid=(N // BT,),
        in_specs=[
            pl.BlockSpec((BT, 3), lambda i: (i, 0)),
            pl.BlockSpec((6, F), lambda i: (0, 0)),
            pl.BlockSpec((3, F), lambda i: (0, 0)),
            pl.BlockSpec((3, F), lambda i: (0, 0)),
            pl.BlockSpec((1, F), lambda i: (0, 0)),
            pl.BlockSpec((1, F), lambda i: (0, 0)),
            pl.BlockSpec((1, F), lambda i: (0, 0)),
        ],
        out_specs=[pl.BlockSpec((BT, F // 2), lambda i: (i, 0))]
        + [pl.BlockSpec((BT, F), lambda i: (i, 0))] * 5,
        out_shape=[outb, outf, outf, outf, outf, outf],
    )(pos, w1a, w2p, w3p, ba1, ba2, ba3)


_NC = 2
_NS = 16
_NW = _NC * _NS
_EDGES = N * K
_PER_W = _EDGES // _NW
_CH = 200
_NCH = _PER_W // _CH


def _sc_gather(table, idx):
    mesh = plsc.VectorSubcoreMesh(core_axis_name="c", subcore_axis_name="s")

    @functools.partial(
        pl.kernel,
        mesh=mesh,
        out_type=jax.ShapeDtypeStruct((_EDGES, F // 2), jnp.int32),
        scratch_types=[
            pltpu.VMEM((_CH,), jnp.int32),
            pltpu.VMEM((_CH, F // 2), jnp.int32),
            pltpu.SemaphoreType.DMA,
        ],
    )
    def gk(idx_hbm, table_hbm, out_hbm, idx_v, rows_v, sem):
        wid = lax.axis_index("s") * _NC + lax.axis_index("c")
        base = wid * _PER_W

        def body(c, carry):
            off = base + c * _CH
            pltpu.sync_copy(idx_hbm.at[pl.ds(off, _CH)], idx_v)
            pltpu.async_copy(table_hbm.at[idx_v], rows_v, sem).wait()
            pltpu.sync_copy(rows_v, out_hbm.at[pl.ds(off, _CH)])
            return carry

        lax.fori_loop(0, _NCH, body, 0)

    return gk(idx, table)


BN = 400


def _mmf(a, w):
    return lax.dot_general(a, w, (((1,), (0,)), ((), ())),
                           preferred_element_type=jnp.float32)


def _conv_core(x_ref, cdst_ref, wb_ref, bb_ref):
    cdst = cdst_ref[...]
    wb = wb_ref[...]
    acc = jnp.full((BN, F), -jnp.inf, jnp.float32)
    for j in range(K):
        z = _unpack_bf16(x_ref[j]) - cdst
        zb = jnp.maximum(z, 0.0).astype(jnp.bfloat16)
        acc = jnp.maximum(acc, _mmf(zb, wb))
    return jnp.maximum(acc + bb_ref[...], 0.0)


def _conv_g_body(x_ref, cdst_ref, wb_ref, bb_ref,
                 wnext_ref, bnext_ref, g_ref):
    h = _conv_core(x_ref, cdst_ref, wb_ref, bb_ref)
    g = _mmf(h.astype(jnp.bfloat16), wnext_ref[...]) + bnext_ref[...]
    g_ref[...] = _pack_bf16(g)


def _conv_g(x, cdst, wb, bb, wnext, bnext):
    return pl.pallas_call(
        _conv_g_body,
        grid=(N // BN,),
        in_specs=[
            pl.BlockSpec((K, BN, F // 2), lambda i: (0, i, 0)),
            pl.BlockSpec((BN, F), lambda i: (i, 0)),
            pl.BlockSpec((F, F), lambda i: (0, 0)),
            pl.BlockSpec((1, F), lambda i: (0, 0)),
            pl.BlockSpec((F, F), lambda i: (0, 0)),
            pl.BlockSpec((BN, F), lambda i: (i, 0)),
        ],
        out_specs=pl.BlockSpec((BN, F // 2), lambda i: (i, 0)),
        out_shape=jax.ShapeDtypeStruct((N, F // 2), jnp.int32),
    )(x, cdst, wb, bb, wnext, bnext)


def _conv_head_body(x_ref, cdst_ref, wb_ref, bb_ref,
                    wh1_ref, bh1_ref, wh2_ref, bh2_ref, wh3_ref, bh3_ref,
                    out_ref):
    h = _conv_core(x_ref, cdst_ref, wb_ref, bb_ref)
    t = jnp.maximum(_mmf(h, wh1_ref[...]) + bh1_ref[...], 0.0)
    t = jnp.maximum(_mmf(t, wh2_ref[...]) + bh2_ref[...], 0.0)
    o = _mmf(t, wh3_ref[...]) + bh3_ref[...]
    out_ref[...] = 1.0 / (1.0 + jnp.exp(-o))


def _conv_head(x, cdst, wb, bb, wh1, bh1, wh2, bh2, wh3, bh3):
    return pl.pallas_call(
        _conv_head_body,
        grid=(N // BN,),
        in_specs=[
            pl.BlockSpec((K, BN, F // 2), lambda i: (0, i, 0)),
            pl.BlockSpec((BN, F), lambda i: (i, 0)),
            pl.BlockSpec((F, F), lambda i: (0, 0)),
            pl.BlockSpec((1, F), lambda i: (0, 0)),
            pl.BlockSpec((F, 128), lambda i: (0, 0)),
            pl.BlockSpec((1, 128), lambda i: (0, 0)),
            pl.BlockSpec((128, 128), lambda i: (0, 0)),
            pl.BlockSpec((1, 128), lambda i: (0, 0)),
            pl.BlockSpec((128, 1), lambda i: (0, 0)),
            pl.BlockSpec((1, 1), lambda i: (0, 0)),
        ],
        out_specs=pl.BlockSpec((BN, 1), lambda i: (i, 0)),
        out_shape=jax.ShapeDtypeStruct((N, 1), jnp.float32),
    )(x, cdst, wb, bb, wh1, bh1, wh2, bh2, wh3, bh3)


def kernel(pos, W1a, b1a, W1b, b1b, W2a, b2a, W2b, b2b, W3a, b3a, W3b, b3b,
           Wh1, bh1, Wh2, bh2, Wh3, bh3):
    post = jnp.pad(pos.T, ((0, 0), (0, NPAD - N)),
                   constant_values=1e15)
    nbr = _knn(pos, post)
    idx = nbr.T.reshape(-1)

    r = lambda b: b.reshape(1, -1)
    bf = lambda w: w.astype(jnp.bfloat16)
    g1, c1, b2t, c2, b3t, c3 = _tables(pos, W1a, W2a[256:, :], W3a[256:, :],
                                       r(b1a), r(b2a), r(b3a))
    x = _sc_gather(g1, idx).reshape(K, N, F // 2)
    g2 = _conv_g(x, c1, bf(W1b), r(b1b), bf(W2a[:256, :]), b2t)
    x = _sc_gather(g2, idx).reshape(K, N, F // 2)
    g3 = _conv_g(x, c2, bf(W2b), r(b2b), bf(W3a[:256, :]), b3t)
    x = _sc_gather(g3, idx).reshape(K, N, F // 2)
    return _conv_head(x, c3, bf(W3b), r(b3b),
                      Wh1, r(bh1), Wh2, r(bh2), Wh3, r(bh3))


# speedup vs baseline: 2.6105x; 2.6105x over previous
import functools

import jax
import jax.numpy as jnp
from jax import lax
from jax.experimental import pallas as pl
from jax.experimental.pallas import tpu as pltpu
from jax.experimental.pallas import tpu_sc as plsc

N = 10000
K = 16
F = 256

BQ = 200
NPAD = 10112
NLPT = 4
_IMAX = 0x7FFFFFFF


def _knn_body(posq_ref, post_ref, out_ref):
    q = posq_ref[...]
    pt = post_ref[...]
    sq = jnp.sum(pt * pt, axis=0, keepdims=True)
    qq = jnp.sum(q * q, axis=1, keepdims=True)
    d2 = qq - 2.0 * lax.dot_general(q, pt, (((1,), (0,)), ((), ())),
                                    preferred_element_type=jnp.float32) + sq
    iota = lax.broadcasted_iota(jnp.int32, (BQ, NPAD), 1)
    bits = lax.bitcast_convert_type(jnp.maximum(d2, 0.0), jnp.int32)
    key = (bits & jnp.int32(-16384)) | iota

    tops = [[jnp.full((BQ, 128), jnp.int32(_IMAX), jnp.int32)
             for _ in range(NLPT)] for _ in range(2)]
    for c in range(NPAD // 128):
        x = key[:, c * 128:(c + 1) * 128]
        tset = tops[c % 2]
        for i in range(NLPT):
            ti = tset[i]
            lo = jnp.minimum(ti, x)
            x = jnp.maximum(ti, x)
            tset[i] = lo

    cand = jnp.concatenate(tops[0] + tops[1], axis=1)
    cols = []
    for _ in range(K):
        m = jnp.min(cand, axis=1, keepdims=True)
        cols.append(m & jnp.int32(16383))
        cand = jnp.where(cand == m, jnp.int32(_IMAX), cand)
    out_ref[...] = jnp.concatenate(cols, axis=1)

    v = m
    tlast = jnp.minimum(tops[0][NLPT - 1], tops[1][NLPT - 1])
    flag = jnp.any(jnp.min(tlast, axis=1, keepdims=True) < v)

    @pl.when(flag)
    def _slow():
        kk = key
        cols2 = []
        for _ in range(K):
            mm = jnp.min(kk, axis=1, keepdims=True)
            cols2.append(mm & jnp.int32(16383))
            kk = jnp.where(kk == mm, jnp.int32(_IMAX), kk)
        out_ref[...] = jnp.concatenate(cols2, axis=1)


def _knn(pos, post):
    return pl.pallas_call(
        _knn_body,
        grid=(N // BQ,),
        in_specs=[
            pl.BlockSpec((BQ, 3), lambda i: (i, 0)),
            pl.BlockSpec((3, NPAD), lambda i: (0, 0)),
        ],
        out_specs=pl.BlockSpec((BQ, K), lambda i: (i, 0)),
        out_shape=jax.ShapeDtypeStruct((N, K), jnp.int32),
    )(pos, post)


def _pack_bf16(g):
    ge = g[:, :128].astype(jnp.bfloat16).astype(jnp.float32)
    go = g[:, 128:].astype(jnp.bfloat16).astype(jnp.float32)
    eb = lax.bitcast_convert_type(ge, jnp.int32)
    ob = lax.bitcast_convert_type(go, jnp.int32)
    return lax.shift_right_logical(eb, 16) | (ob & jnp.int32(-65536))


def _unpack_bf16(w):
    even = lax.bitcast_convert_type(lax.shift_left(w, 16), jnp.float32)
    odd = lax.bitcast_convert_type(w & jnp.int32(-65536), jnp.float32)
    return jnp.concatenate([even, odd], axis=1)


BT = 400


def _tables_body(pos_ref, w1_ref, w2p_ref, w3p_ref, ba1_ref, ba2_ref, ba3_ref,
                 g1_ref, c1_ref, b2_ref, c2_ref, b3_ref, c3_ref):
    p = pos_ref[...]

    def mm(w):
        return lax.dot_general(p, w, (((1,), (0,)), ((), ())),
                               preferred_element_type=jnp.float32)

    w1 = w1_ref[...]
    b1 = mm(w1[3:6, :])
    g1_ref[...] = _pack_bf16(mm(w1[0:3, :]) + b1)
    c1_ref[...] = b1 - ba1_ref[...]
    b2 = mm(w2p_ref[...])
    b2_ref[...] = b2
    c2_ref[...] = b2 - ba2_ref[...]
    b3 = mm(w3p_ref[...])
    b3_ref[...] = b3
    c3_ref[...] = b3 - ba3_ref[...]


def _tables(pos, w1a, w2p, w3p, ba1, ba2, ba3):
    outf = jax.ShapeDtypeStruct((N, F), jnp.float32)
    outb = jax.ShapeDtypeStruct((N, F // 2), jnp.int32)
    return pl.pallas_call(
        _tables_body,
        grid=(N // BT,),
        in_specs=[
            pl.BlockSpec((BT, 3), lambda i: (i, 0)),
            pl.BlockSpec((6, F), lambda i: (0, 0)),
            pl.BlockSpec((3, F), lambda i: (0, 0)),
            pl.BlockSpec((3, F), lambda i: (0, 0)),
            pl.BlockSpec((1, F), lambda i: (0, 0)),
            pl.BlockSpec((1, F), lambda i: (0, 0)),
            pl.BlockSpec((1, F), lambda i: (0, 0)),
        ],
        out_specs=[pl.BlockSpec((BT, F // 2), lambda i: (i, 0))]
        + [pl.BlockSpec((BT, F), lambda i: (i, 0))] * 5,
        out_shape=[outb, outf, outf, outf, outf, outf],
    )(pos, w1a, w2p, w3p, ba1, ba2, ba3)


_NC = 2
_NS = 16
_NW = _NC * _NS
_EDGES = N * K
_PER_W = _EDGES // _NW
_CH = 200
_NCH = _PER_W // _CH


def _sc_gather(table, idx):
    mesh = plsc.VectorSubcoreMesh(core_axis_name="c", subcore_axis_name="s")

    @functools.partial(
        pl.kernel,
        mesh=mesh,
        out_type=jax.ShapeDtypeStruct((_EDGES, F // 2), jnp.int32),
        scratch_types=[
            pltpu.VMEM((_CH,), jnp.int32),
            pltpu.VMEM((_CH, F // 2), jnp.int32),
            pltpu.SemaphoreType.DMA,
        ],
    )
    def gk(idx_hbm, table_hbm, out_hbm, idx_v, rows_v, sem):
        wid = lax.axis_index("s") * _NC + lax.axis_index("c")
        base = wid * _PER_W

        def body(c, carry):
            off = base + c * _CH
            pltpu.sync_copy(idx_hbm.at[pl.ds(off, _CH)], idx_v)
            pltpu.async_copy(table_hbm.at[idx_v], rows_v, sem).wait()
            pltpu.sync_copy(rows_v, out_hbm.at[pl.ds(off, _CH)])
            return carry

        lax.fori_loop(0, _NCH, body, 0)

    return gk(idx, table)


BN = 400


def _mmf(a, w):
    return lax.dot_general(a, w, (((1,), (0,)), ((), ())),
                           preferred_element_type=jnp.float32)


def _conv_core(x_ref, cdst_ref, wb_ref, bb_ref):
    cdst = cdst_ref[...]
    wb = wb_ref[...]
    acc = jnp.full((BN, F), -jnp.inf, jnp.float32)
    for j in range(K):
        z = _unpack_bf16(x_ref[j]) - cdst
        zb = jnp.maximum(z, 0.0).astype(jnp.bfloat16)
        acc = jnp.maximum(acc, _mmf(zb, wb))
    return jnp.maximum(acc + bb_ref[...], 0.0)


def _conv_g_body(x_ref, cdst_ref, wb_ref, bb_ref,
                 wnext_ref, bnext_ref, g_ref):
    h = _conv_core(x_ref, cdst_ref, wb_ref, bb_ref)
    g = _mmf(h.astype(jnp.bfloat16), wnext_ref[...]) + bnext_ref[...]
    g_ref[...] = _pack_bf16(g)


def _conv_g(x, cdst, wb, bb, wnext, bnext):
    return pl.pallas_call(
        _conv_g_body,
        grid=(N // BN,),
        in_specs=[
            pl.BlockSpec((K, BN, F // 2), lambda i: (0, i, 0)),
            pl.BlockSpec((BN, F), lambda i: (i, 0)),
            pl.BlockSpec((F, F), lambda i: (0, 0)),
            pl.BlockSpec((1, F), lambda i: (0, 0)),
            pl.BlockSpec((F, F), lambda i: (0, 0)),
            pl.BlockSpec((BN, F), lambda i: (i, 0)),
        ],
        out_specs=pl.BlockSpec((BN, F // 2), lambda i: (i, 0)),
        out_shape=jax.ShapeDtypeStruct((N, F // 2), jnp.int32),
    )(x, cdst, wb, bb, wnext, bnext)


def _conv_head_body(x_ref, cdst_ref, wb_ref, bb_ref,
                    wh1_ref, bh1_ref, wh2_ref, bh2_ref, wh3_ref, bh3_ref,
                    out_ref):
    h = _conv_core(x_ref, cdst_ref, wb_ref, bb_ref)
    t = jnp.maximum(_mmf(h, wh1_ref[...]) + bh1_ref[...], 0.0)
    t = jnp.maximum(_mmf(t, wh2_ref[...]) + bh2_ref[...], 0.0)
    o = _mmf(t, wh3_ref[...]) + bh3_ref[...]
    out_ref[...] = 1.0 / (1.0 + jnp.exp(-o))


def _conv_head(x, cdst, wb, bb, wh1, bh1, wh2, bh2, wh3, bh3):
    return pl.pallas_call(
        _conv_head_body,
        grid=(N // BN,),
        in_specs=[
            pl.BlockSpec((K, BN, F // 2), lambda i: (0, i, 0)),
            pl.BlockSpec((BN, F), lambda i: (i, 0)),
            pl.BlockSpec((F, F), lambda i: (0, 0)),
            pl.BlockSpec((1, F), lambda i: (0, 0)),
            pl.BlockSpec((F, 128), lambda i: (0, 0)),
            pl.BlockSpec((1, 128), lambda i: (0, 0)),
            pl.BlockSpec((128, 128), lambda i: (0, 0)),
            pl.BlockSpec((1, 128), lambda i: (0, 0)),
            pl.BlockSpec((128, 1), lambda i: (0, 0)),
            pl.BlockSpec((1, 1), lambda i: (0, 0)),
        ],
        out_specs=pl.BlockSpec((BN, 1), lambda i: (i, 0)),
        out_shape=jax.ShapeDtypeStruct((N, 1), jnp.float32),
    )(x, cdst, wb, bb, wh1, bh1, wh2, bh2, wh3, bh3)


def kernel(pos, W1a, b1a, W1b, b1b, W2a, b2a, W2b, b2b, W3a, b3a, W3b, b3b,
           Wh1, bh1, Wh2, bh2, Wh3, bh3):
    post = jnp.pad(pos.T, ((0, 0), (0, NPAD - N)),
                   constant_values=1e15)
    nbr = _knn(pos, post)
    idx = nbr.T.reshape(-1)

    r = lambda b: b.reshape(1, -1)
    bf = lambda w: w.astype(jnp.bfloat16)
    g1, c1, b2t, c2, b3t, c3 = _tables(pos, W1a, W2a[256:, :], W3a[256:, :],
                                       r(b1a), r(b2a), r(b3a))
    x = _sc_gather(g1, idx).reshape(K, N, F // 2)
    g2 = _conv_g(x, c1, bf(W1b), r(b1b), bf(W2a[:256, :]), b2t)
    x = _sc_gather(g2, idx).reshape(K, N, F // 2)
    g3 = _conv_g(x, c2, bf(W2b), r(b2b), bf(W3a[:256, :]), b3t)
    x = _sc_gather(g3, idx).reshape(K, N, F // 2)
    return _conv_head(x, c3, bf(W3b), r(b3b),
                      Wh1, r(bh1), Wh2, r(bh2), Wh3, r(bh3))
